# Initial kernel scaffold; baseline (speedup 1.0000x reference)
import jax
import jax.numpy as jnp
from jax.experimental import pallas as pl


def kernel(texts, table, W1, b1, W2, b2):
    raise NotImplementedError("write your pallas kernel here")


# speedup vs baseline: 1.9511x; 1.9511x over previous
import functools

import jax
import jax.numpy as jnp
from jax import lax
from jax.experimental import pallas as pl
from jax.experimental.pallas import tpu as pltpu
from jax.experimental.pallas import tpu_sc as plsc

VOCAB = 1000000
EMBED = 32
HIDDEN = 64
OUT_VOCAB = 1000
BATCH = 4096
HIST = 200

NC = 2
NS = 16
NW = NC * NS
B_PER_W = BATCH // NW
CB = 4
NCHUNK = B_PER_W // CB
IDX_SUB = 8
IDX_LEN = CB * HIST // IDX_SUB


def _pool_body(texts_hbm, table_hbm, out_hbm, idx_v, rows_v, acc_v, sem):
    wid = lax.axis_index("s") * NC + lax.axis_index("c")
    inv = jnp.float32(1.0 / HIST)

    def chunk_body(g, _):
        cid = wid * NCHUNK + g
        pltpu.sync_copy(texts_hbm.at[cid], idx_v)
        copies = []
        for j in range(IDX_SUB):
            copies.append(
                pltpu.async_copy(
                    table_hbm.at[idx_v.at[j]],
                    rows_v.at[pl.ds(j * IDX_LEN, IDX_LEN)],
                    sem,
                )
            )
        for c in copies:
            c.wait()

        for c in range(CB):
            def sum_body(i, carry):
                a0, a1 = carry
                r = c * HIST + i
                return a0 + rows_v[r, 0:16], a1 + rows_v[r, 16:32]

            a0, a1 = lax.fori_loop(
                0, HIST, sum_body,
                (jnp.zeros((16,), jnp.float32), jnp.zeros((16,), jnp.float32)),
            )
            acc_v[c, 0:16] = a0 * inv
            acc_v[c, 16:32] = a1 * inv

        pltpu.sync_copy(acc_v, out_hbm.at[pl.ds(cid * CB, CB)])
        return ()

    lax.fori_loop(0, NCHUNK, chunk_body, ())


def _sc_pool(texts, table):
    mesh = plsc.VectorSubcoreMesh(core_axis_name="c", subcore_axis_name="s")
    texts_r = texts.reshape(NW * NCHUNK, IDX_SUB, IDX_LEN)
    f = pl.kernel(
        _pool_body,
        mesh=mesh,
        compiler_params=pltpu.CompilerParams(use_tc_tiling_on_sc=False),
        out_type=jax.ShapeDtypeStruct((BATCH, EMBED), jnp.float32),
        scratch_types=[
            pltpu.VMEM((IDX_SUB, IDX_LEN), jnp.int32),
            pltpu.VMEM((CB * HIST, EMBED), jnp.float32),
            pltpu.VMEM((CB, EMBED), jnp.float32),
            pltpu.SemaphoreType.DMA,
        ],
    )
    return f(texts_r, table)


def _mlp_body(x_ref, w1_ref, b1_ref, w2_ref, b2_ref, o_ref):
    h = jnp.dot(x_ref[...], w1_ref[...], preferred_element_type=jnp.float32)
    h = jnp.maximum(h + b1_ref[...], 0.0)
    o = jnp.dot(h, w2_ref[...], preferred_element_type=jnp.float32)
    o_ref[...] = o + b2_ref[...]


def _tc_mlp(pooled, W1, b1, W2, b2):
    BM = 512
    grid = (BATCH // BM,)
    return pl.pallas_call(
        _mlp_body,
        grid=grid,
        in_specs=[
            pl.BlockSpec((BM, EMBED), lambda i: (i, 0)),
            pl.BlockSpec((EMBED, HIDDEN), lambda i: (0, 0)),
            pl.BlockSpec((1, HIDDEN), lambda i: (0, 0)),
            pl.BlockSpec((HIDDEN, OUT_VOCAB), lambda i: (0, 0)),
            pl.BlockSpec((1, OUT_VOCAB), lambda i: (0, 0)),
        ],
        out_specs=pl.BlockSpec((BM, OUT_VOCAB), lambda i: (i, 0)),
        out_shape=jax.ShapeDtypeStruct((BATCH, OUT_VOCAB), jnp.float32),
    )(pooled, W1, b1.reshape(1, HIDDEN), W2, b2.reshape(1, OUT_VOCAB))


def kernel(texts, table, W1, b1, W2, b2):
    pooled = _sc_pool(texts.astype(jnp.int32), table)
    return _tc_mlp(pooled, W1, b1, W2, b2)


# speedup vs baseline: 1.9583x; 1.0037x over previous
import functools

import jax
import jax.numpy as jnp
from jax import lax
from jax.experimental import pallas as pl
from jax.experimental.pallas import tpu as pltpu
from jax.experimental.pallas import tpu_sc as plsc

VOCAB = 1000000
EMBED = 32
HIDDEN = 64
OUT_VOCAB = 1000
BATCH = 4096
HIST = 200

NC = 2
NS = 16
NW = NC * NS
B_PER_W = BATCH // NW
CB = 4
NCHUNK = B_PER_W // CB
IDX_SPLITS = ((0, 128), (128, 72))


def _pool_body(texts_hbm, table_hbm, out_hbm, idx_v, rows_v, acc_v, sem):
    wid = lax.axis_index("s") * NC + lax.axis_index("c")
    inv = jnp.float32(1.0 / HIST)

    def chunk_body(g, _):
        cid = wid * NCHUNK + g
        pltpu.sync_copy(texts_hbm.at[pl.ds(cid * CB, CB)], idx_v)
        copies = []
        for c in range(CB):
            for off, ln in IDX_SPLITS:
                copies.append(
                    pltpu.async_copy(
                        table_hbm.at[idx_v.at[c, pl.ds(off, ln)]],
                        rows_v.at[pl.ds(c * HIST + off, ln)],
                        sem,
                    )
                )
        for c in copies:
            c.wait()

        for c in range(CB):
            def sum_body(i, carry):
                a0, a1 = carry
                r = c * HIST + i
                return a0 + rows_v[r, 0:16], a1 + rows_v[r, 16:32]

            a0, a1 = lax.fori_loop(
                0, HIST, sum_body,
                (jnp.zeros((16,), jnp.float32), jnp.zeros((16,), jnp.float32)),
            )
            acc_v[c, 0:16] = a0 * inv
            acc_v[c, 16:32] = a1 * inv

        pltpu.sync_copy(acc_v, out_hbm.at[pl.ds(cid * CB, CB)])
        return ()

    lax.fori_loop(0, NCHUNK, chunk_body, ())


def _sc_pool(texts, table):
    mesh = plsc.VectorSubcoreMesh(core_axis_name="c", subcore_axis_name="s")
    f = pl.kernel(
        _pool_body,
        mesh=mesh,
        compiler_params=pltpu.CompilerParams(use_tc_tiling_on_sc=False),
        out_type=jax.ShapeDtypeStruct((BATCH, EMBED), jnp.float32),
        scratch_types=[
            pltpu.VMEM((CB, HIST), jnp.int32),
            pltpu.VMEM((CB * HIST, EMBED), jnp.float32),
            pltpu.VMEM((CB, EMBED), jnp.float32),
            pltpu.SemaphoreType.DMA,
        ],
    )
    return f(texts, table)


def _mlp_body(x_ref, w1_ref, b1_ref, w2_ref, b2_ref, o_ref):
    h = jnp.dot(x_ref[...], w1_ref[...], preferred_element_type=jnp.float32)
    h = jnp.maximum(h + b1_ref[...], 0.0)
    o = jnp.dot(h, w2_ref[...], preferred_element_type=jnp.float32)
    o_ref[...] = o + b2_ref[...]


def _tc_mlp(pooled, W1, b1, W2, b2):
    BM = 512
    grid = (BATCH // BM,)
    return pl.pallas_call(
        _mlp_body,
        grid=grid,
        in_specs=[
            pl.BlockSpec((BM, EMBED), lambda i: (i, 0)),
            pl.BlockSpec((EMBED, HIDDEN), lambda i: (0, 0)),
            pl.BlockSpec((1, HIDDEN), lambda i: (0, 0)),
            pl.BlockSpec((HIDDEN, OUT_VOCAB), lambda i: (0, 0)),
            pl.BlockSpec((1, OUT_VOCAB), lambda i: (0, 0)),
        ],
        out_specs=pl.BlockSpec((BM, OUT_VOCAB), lambda i: (i, 0)),
        out_shape=jax.ShapeDtypeStruct((BATCH, OUT_VOCAB), jnp.float32),
    )(pooled, W1, b1.reshape(1, HIDDEN), W2, b2.reshape(1, OUT_VOCAB))


def kernel(texts, table, W1, b1, W2, b2):
    pooled = _sc_pool(texts.astype(jnp.int32), table)
    return _tc_mlp(pooled, W1, b1, W2, b2)


# speedup vs baseline: 2.2965x; 1.1727x over previous
import functools

import jax
import jax.numpy as jnp
from jax import lax
from jax.experimental import pallas as pl
from jax.experimental.pallas import tpu as pltpu
from jax.experimental.pallas import tpu_sc as plsc

VOCAB = 1000000
EMBED = 32
HIDDEN = 64
OUT_VOCAB = 1000
BATCH = 4096
HIST = 200

NC = 2
NS = 16
NW = NC * NS
B_PER_W = BATCH // NW
CB = 4
NCHUNK = B_PER_W // CB
IDX_SPLITS = ((0, 128), (128, 72))


def _pool_body(texts_hbm, table_hbm, out_hbm, idx_v, rows_v, acc_v, sem):
    wid = lax.axis_index("s") * NC + lax.axis_index("c")
    inv = jnp.float32(1.0 / HIST)

    def chunk_body(g, _):
        cid = wid * NCHUNK + g
        pltpu.sync_copy(texts_hbm.at[pl.ds(cid * CB, CB)], idx_v)
        copies = []
        for c in range(CB):
            for off, ln in IDX_SPLITS:
                copies.append(
                    pltpu.async_copy(
                        table_hbm.at[idx_v.at[c, pl.ds(off, ln)]],
                        rows_v.at[pl.ds(c * HIST + off, ln)],
                        sem,
                    )
                )
        for c in copies:
            c.wait()

        for c in range(CB):
            def sum_body(i, carry):
                a0, a1 = carry
                r = c * HIST + i
                return a0 + rows_v[r, 0:16], a1 + rows_v[r, 16:32]

            a0, a1 = lax.fori_loop(
                0, HIST, sum_body,
                (jnp.zeros((16,), jnp.float32), jnp.zeros((16,), jnp.float32)),
            )
            acc_v[c, 0:16] = a0 * inv
            acc_v[c, 16:32] = a1 * inv

        pltpu.sync_copy(acc_v, out_hbm.at[pl.ds(cid * CB, CB)])
        return ()

    lax.fori_loop(0, NCHUNK, chunk_body, ())


def _sc_pool(texts, table):
    mesh = plsc.VectorSubcoreMesh(core_axis_name="c", subcore_axis_name="s")
    f = pl.kernel(
        _pool_body,
        mesh=mesh,
        compiler_params=pltpu.CompilerParams(use_tc_tiling_on_sc=False),
        out_type=jax.ShapeDtypeStruct((BATCH, EMBED), jnp.float32),
        scratch_types=[
            pltpu.VMEM((CB, HIST), jnp.int32),
            pltpu.VMEM((CB * HIST, EMBED), jnp.float32),
            pltpu.VMEM((CB, EMBED), jnp.float32),
            pltpu.SemaphoreType.DMA,
        ],
    )
    return f(texts, table)


RELAYOUT_VB = 4096


def _relayout_body(src_ref, out_ref):
    y = jnp.transpose(src_ref[...], (1, 0))
    y3 = y.reshape(RELAYOUT_VB // 4, 4, EMBED)
    for a in range(4):
        out_ref[:, EMBED * a:EMBED * (a + 1)] = y3[:, a, :]


def _tc_relayout(table):
    grid = ((VOCAB + RELAYOUT_VB - 1) // RELAYOUT_VB,)
    packed = pl.pallas_call(
        _relayout_body,
        grid=grid,
        in_specs=[pl.BlockSpec((EMBED, RELAYOUT_VB), lambda i: (0, i))],
        out_specs=pl.BlockSpec((RELAYOUT_VB // 4, 128), lambda i: (i, 0)),
        out_shape=jax.ShapeDtypeStruct((VOCAB // 4, 128), jnp.float32),
    )(table.T)
    return packed.reshape(VOCAB, EMBED)


def _mlp_body(x_ref, w1_ref, b1_ref, w2_ref, b2_ref, o_ref):
    h = jnp.dot(x_ref[...], w1_ref[...], preferred_element_type=jnp.float32)
    h = jnp.maximum(h + b1_ref[...], 0.0)
    o = jnp.dot(h, w2_ref[...], preferred_element_type=jnp.float32)
    o_ref[...] = o + b2_ref[...]


def _tc_mlp(pooled, W1, b1, W2, b2):
    BM = 512
    grid = (BATCH // BM,)
    return pl.pallas_call(
        _mlp_body,
        grid=grid,
        in_specs=[
            pl.BlockSpec((BM, EMBED), lambda i: (i, 0)),
            pl.BlockSpec((EMBED, HIDDEN), lambda i: (0, 0)),
            pl.BlockSpec((1, HIDDEN), lambda i: (0, 0)),
            pl.BlockSpec((HIDDEN, OUT_VOCAB), lambda i: (0, 0)),
            pl.BlockSpec((1, OUT_VOCAB), lambda i: (0, 0)),
        ],
        out_specs=pl.BlockSpec((BM, OUT_VOCAB), lambda i: (i, 0)),
        out_shape=jax.ShapeDtypeStruct((BATCH, OUT_VOCAB), jnp.float32),
    )(pooled, W1, b1.reshape(1, HIDDEN), W2, b2.reshape(1, OUT_VOCAB))


def kernel(texts, table, W1, b1, W2, b2):
    table_lin = _tc_relayout(table)
    pooled = _sc_pool(texts.astype(jnp.int32), table_lin)
    return _tc_mlp(pooled, W1, b1, W2, b2)


# speedup vs baseline: 3.7885x; 1.6497x over previous
import functools

import jax
import jax.numpy as jnp
from jax import lax
from jax.experimental import pallas as pl
from jax.experimental.pallas import tpu as pltpu
from jax.experimental.pallas import tpu_sc as plsc

VOCAB = 1000000
EMBED = 32
HIDDEN = 64
OUT_VOCAB = 1000
BATCH = 4096
HIST = 200

PACK_S = 1 << 18
PACK_SHIFT = 18
PACK_MASK = PACK_S - 1
VIEW_ROWS = 4 * PACK_S

NC = 2
NS = 16
NW = NC * NS
B_PER_W = BATCH // NW
CB = 4
NCHUNK = B_PER_W // CB
CHUNK_IDX = CB * HIST
IDX_SPLITS = ((0, 128), (128, 72))


def _pool_body(texts_hbm, table_hbm, out_hbm, idx_v, rows_v, acc_v, sem):
    wid = lax.axis_index("s") * NC + lax.axis_index("c")
    inv = jnp.float32(1.0 / HIST)

    def chunk_body(g, _):
        cid = wid * NCHUNK + g
        pltpu.sync_copy(texts_hbm.at[pl.ds(cid * CHUNK_IDX, CHUNK_IDX)], idx_v)
        for t in range(CHUNK_IDX // 16):
            v = idx_v[pl.ds(16 * t, 16)]
            idx_v[pl.ds(16 * t, 16)] = ((v & PACK_MASK) << 2) | (v >> PACK_SHIFT)
        copies = []
        for c in range(CB):
            for off, ln in IDX_SPLITS:
                copies.append(
                    pltpu.async_copy(
                        table_hbm.at[idx_v.at[pl.ds(c * HIST + off, ln)]],
                        rows_v.at[pl.ds(c * HIST + off, ln)],
                        sem,
                    )
                )
        for c in copies:
            c.wait()

        for c in range(CB):
            def sum_body(i, carry):
                a0, a1 = carry
                r = c * HIST + i
                return a0 + rows_v[r, 0:16], a1 + rows_v[r, 16:32]

            a0, a1 = lax.fori_loop(
                0, HIST, sum_body,
                (jnp.zeros((16,), jnp.float32), jnp.zeros((16,), jnp.float32)),
            )
            acc_v[c, 0:16] = a0 * inv
            acc_v[c, 16:32] = a1 * inv

        pltpu.sync_copy(acc_v, out_hbm.at[pl.ds(cid * CB, CB)])
        return ()

    lax.fori_loop(0, NCHUNK, chunk_body, ())


def _sc_pool(texts_flat, table_view):
    mesh = plsc.VectorSubcoreMesh(core_axis_name="c", subcore_axis_name="s")
    f = pl.kernel(
        _pool_body,
        mesh=mesh,
        compiler_params=pltpu.CompilerParams(use_tc_tiling_on_sc=False),
        out_type=jax.ShapeDtypeStruct((BATCH, EMBED), jnp.float32),
        scratch_types=[
            pltpu.VMEM((CHUNK_IDX,), jnp.int32),
            pltpu.VMEM((CHUNK_IDX, EMBED), jnp.float32),
            pltpu.VMEM((CB, EMBED), jnp.float32),
            pltpu.SemaphoreType.DMA,
        ],
    )
    return f(texts_flat, table_view)


RL_VB = 2048
RL_QBLK = PACK_S // RL_VB
RL_INBLKS = (VOCAB + RL_VB - 1) // RL_VB


def _relayout_body(s0, s1, s2, s3, out_ref):
    x4 = jnp.concatenate([s0[...], s1[...], s2[...], s3[...]], axis=0)
    out_ref[...] = jnp.transpose(x4, (1, 0))


def _quarter_spec(a):
    def imap(i):
        return (0, jnp.minimum(a * RL_QBLK + i, RL_INBLKS - 1))
    return pl.BlockSpec((EMBED, RL_VB), imap)


def _tc_relayout(table):
    packed = pl.pallas_call(
        _relayout_body,
        grid=(RL_QBLK,),
        in_specs=[_quarter_spec(a) for a in range(4)],
        out_specs=pl.BlockSpec((RL_VB, 4 * EMBED), lambda i: (i, 0)),
        out_shape=jax.ShapeDtypeStruct((PACK_S, 4 * EMBED), jnp.float32),
    )(table.T, table.T, table.T, table.T)
    return packed.reshape(VIEW_ROWS, EMBED)


def _mlp_body(x_ref, w1_ref, b1_ref, w2_ref, b2_ref, o_ref):
    h = jnp.dot(x_ref[...], w1_ref[...], preferred_element_type=jnp.float32)
    h = jnp.maximum(h + b1_ref[...], 0.0)
    o = jnp.dot(h, w2_ref[...], preferred_element_type=jnp.float32)
    o_ref[...] = o + b2_ref[...]


def _tc_mlp(pooled, W1, b1, W2, b2):
    BM = 512
    grid = (BATCH // BM,)
    return pl.pallas_call(
        _mlp_body,
        grid=grid,
        in_specs=[
            pl.BlockSpec((BM, EMBED), lambda i: (i, 0)),
            pl.BlockSpec((EMBED, HIDDEN), lambda i: (0, 0)),
            pl.BlockSpec((1, HIDDEN), lambda i: (0, 0)),
            pl.BlockSpec((HIDDEN, OUT_VOCAB), lambda i: (0, 0)),
            pl.BlockSpec((1, OUT_VOCAB), lambda i: (0, 0)),
        ],
        out_specs=pl.BlockSpec((BM, OUT_VOCAB), lambda i: (i, 0)),
        out_shape=jax.ShapeDtypeStruct((BATCH, OUT_VOCAB), jnp.float32),
    )(pooled, W1, b1.reshape(1, HIDDEN), W2, b2.reshape(1, OUT_VOCAB))


def kernel(texts, table, W1, b1, W2, b2):
    table_view = _tc_relayout(table)
    texts_flat = texts.astype(jnp.int32).reshape(BATCH * HIST)
    pooled = _sc_pool(texts_flat, table_view)
    return _tc_mlp(pooled, W1, b1, W2, b2)


# speedup vs baseline: 5.3494x; 1.4120x over previous
import functools

import jax
import jax.numpy as jnp
from jax import lax
from jax.experimental import pallas as pl
from jax.experimental.pallas import tpu as pltpu
from jax.experimental.pallas import tpu_sc as plsc

VOCAB = 1000000
EMBED = 32
HIDDEN = 64
OUT_VOCAB = 1000
BATCH = 4096
HIST = 200

PACK_S = 1 << 18
PACK_SHIFT = 18
PACK_MASK = PACK_S - 1
VIEW_ROWS = 4 * PACK_S

NC = 2
NS = 16
NW = NC * NS
B_PER_W = BATCH // NW
CB = 4
NCHUNK = B_PER_W // CB
CHUNK_IDX = CB * HIST
IDX_SPLITS = ((0, 128), (128, 72))


def _pool_body(texts_hbm, table_hbm, out_hbm,
               idx0, idx1, rows0, rows1, acc_v, sem0, sem1):
    wid = lax.axis_index("s") * NC + lax.axis_index("c")
    inv = jnp.float32(1.0 / HIST)
    idx_bufs = (idx0, idx1)
    rows_bufs = (rows0, rows1)
    sems = (sem0, sem1)

    def fire(g, buf):
        idx_v, rows_v, sem = idx_bufs[buf], rows_bufs[buf], sems[buf]
        cid = wid * NCHUNK + g
        pltpu.sync_copy(texts_hbm.at[pl.ds(cid * CHUNK_IDX, CHUNK_IDX)], idx_v)
        for t in range(CHUNK_IDX // 16):
            v = idx_v[pl.ds(16 * t, 16)]
            idx_v[pl.ds(16 * t, 16)] = ((v & PACK_MASK) << 2) | (v >> PACK_SHIFT)
        for c in range(CB):
            for off, ln in IDX_SPLITS:
                pltpu.async_copy(
                    table_hbm.at[idx_v.at[pl.ds(c * HIST + off, ln)]],
                    rows_v.at[pl.ds(c * HIST + off, ln)],
                    sem,
                )

    def drain(buf):
        rows_v, sem = rows_bufs[buf], sems[buf]
        for c in range(CB):
            for off, ln in IDX_SPLITS:
                pltpu.make_async_copy(
                    table_hbm.at[idx_bufs[buf].at[pl.ds(c * HIST + off, ln)]],
                    rows_v.at[pl.ds(c * HIST + off, ln)],
                    sem,
                ).wait()

    def compute(g, buf):
        rows_v = rows_bufs[buf]
        cid = wid * NCHUNK + g
        for c in range(CB):
            def sum_body(i, carry):
                a0, a1 = carry
                r = c * HIST + i * 4
                for u in range(4):
                    a0 = a0 + rows_v[r + u, 0:16]
                    a1 = a1 + rows_v[r + u, 16:32]
                return a0, a1

            a0, a1 = lax.fori_loop(
                0, HIST // 4, sum_body,
                (jnp.zeros((16,), jnp.float32), jnp.zeros((16,), jnp.float32)),
            )
            acc_v[c, 0:16] = a0 * inv
            acc_v[c, 16:32] = a1 * inv
        pltpu.sync_copy(acc_v, out_hbm.at[pl.ds(cid * CB, CB)])

    fire(0, 0)

    def pair_body(g2, _):
        g = g2 * 2

        @pl.when(g + 1 < NCHUNK)
        def _():
            fire(g + 1, 1)

        drain(0)
        compute(g, 0)

        @pl.when(g + 2 < NCHUNK)
        def _():
            fire(g + 2, 0)

        drain(1)
        compute(g + 1, 1)
        return ()

    lax.fori_loop(0, NCHUNK // 2, pair_body, ())


def _sc_pool(texts_flat, table_view):
    mesh = plsc.VectorSubcoreMesh(core_axis_name="c", subcore_axis_name="s")
    f = pl.kernel(
        _pool_body,
        mesh=mesh,
        compiler_params=pltpu.CompilerParams(use_tc_tiling_on_sc=False),
        out_type=jax.ShapeDtypeStruct((BATCH, EMBED), jnp.float32),
        scratch_types=[
            pltpu.VMEM((CHUNK_IDX,), jnp.int32),
            pltpu.VMEM((CHUNK_IDX,), jnp.int32),
            pltpu.VMEM((CHUNK_IDX, EMBED), jnp.float32),
            pltpu.VMEM((CHUNK_IDX, EMBED), jnp.float32),
            pltpu.VMEM((CB, EMBED), jnp.float32),
            pltpu.SemaphoreType.DMA,
            pltpu.SemaphoreType.DMA,
        ],
    )
    return f(texts_flat, table_view)


RL_VB = 2048
RL_QBLK = PACK_S // RL_VB
RL_INBLKS = (VOCAB + RL_VB - 1) // RL_VB


def _relayout_body(s0, s1, s2, s3, out_ref):
    x4 = jnp.concatenate([s0[...], s1[...], s2[...], s3[...]], axis=0)
    out_ref[...] = jnp.transpose(x4, (1, 0))


def _quarter_spec(a):
    def imap(i):
        return (0, jnp.minimum(a * RL_QBLK + i, RL_INBLKS - 1))
    return pl.BlockSpec((EMBED, RL_VB), imap)


def _tc_relayout(table):
    packed = pl.pallas_call(
        _relayout_body,
        grid=(RL_QBLK,),
        in_specs=[_quarter_spec(a) for a in range(4)],
        out_specs=pl.BlockSpec((RL_VB, 4 * EMBED), lambda i: (i, 0)),
        out_shape=jax.ShapeDtypeStruct((PACK_S, 4 * EMBED), jnp.float32),
    )(table.T, table.T, table.T, table.T)
    return packed.reshape(VIEW_ROWS, EMBED)


def _mlp_body(x_ref, w1_ref, b1_ref, w2_ref, b2_ref, o_ref):
    h = jnp.dot(x_ref[...], w1_ref[...], preferred_element_type=jnp.float32)
    h = jnp.maximum(h + b1_ref[...], 0.0)
    o = jnp.dot(h, w2_ref[...], preferred_element_type=jnp.float32)
    o_ref[...] = o + b2_ref[...]


def _tc_mlp(pooled, W1, b1, W2, b2):
    BM = 512
    grid = (BATCH // BM,)
    return pl.pallas_call(
        _mlp_body,
        grid=grid,
        in_specs=[
            pl.BlockSpec((BM, EMBED), lambda i: (i, 0)),
            pl.BlockSpec((EMBED, HIDDEN), lambda i: (0, 0)),
            pl.BlockSpec((1, HIDDEN), lambda i: (0, 0)),
            pl.BlockSpec((HIDDEN, OUT_VOCAB), lambda i: (0, 0)),
            pl.BlockSpec((1, OUT_VOCAB), lambda i: (0, 0)),
        ],
        out_specs=pl.BlockSpec((BM, OUT_VOCAB), lambda i: (i, 0)),
        out_shape=jax.ShapeDtypeStruct((BATCH, OUT_VOCAB), jnp.float32),
    )(pooled, W1, b1.reshape(1, HIDDEN), W2, b2.reshape(1, OUT_VOCAB))


def kernel(texts, table, W1, b1, W2, b2):
    table_view = _tc_relayout(table)
    texts_flat = texts.astype(jnp.int32).reshape(BATCH * HIST)
    pooled = _sc_pool(texts_flat, table_view)
    return _tc_mlp(pooled, W1, b1, W2, b2)
